# baseline (speedup 1.0000x reference)
import jax
import jax.numpy as jnp
from jax.experimental import pallas as pl
from jax.experimental.pallas import tpu as pltpu

N = 4096
F = 64
BM = 256
NSTEPS = N // BM
NBUF = 4


def _fused_kernel(b_hbm, adj_hbm, w_hbm, bias_hbm, out_ref,
                  buf_ref, sup_ref, b_ref, w_ref, bias_ref,
                  sem_ref, sem_small):
    i = pl.program_id(0)

    def _copy(block, slot):
        return pltpu.make_async_copy(
            adj_hbm.at[pl.ds(block * BM, BM), :],
            buf_ref.at[slot],
            sem_ref.at[slot],
        )

    @pl.when(i == 0)
    def _():
        for j in range(NBUF):
            _copy(j, j).start()
        cb = pltpu.make_async_copy(b_hbm, b_ref, sem_small.at[0])
        cw = pltpu.make_async_copy(w_hbm, w_ref, sem_small.at[1])
        cs = pltpu.make_async_copy(bias_hbm, bias_ref, sem_small.at[2])
        cb.start()
        cw.start()
        cs.start()
        cb.wait()
        cw.wait()
        cs.wait()
        sup_ref[...] = jnp.dot(
            b_ref[...], w_ref[...], preferred_element_type=jnp.float32
        ).astype(jnp.bfloat16)

    slot = jax.lax.rem(i, NBUF)
    _copy(i, slot).wait()
    adj_bf = buf_ref[slot].astype(jnp.bfloat16)
    out_ref[...] = (
        jnp.dot(adj_bf, sup_ref[...], preferred_element_type=jnp.float32)
        + bias_ref[...]
    )

    nxt = i + NBUF

    @pl.when(nxt < NSTEPS)
    def _():
        _copy(nxt, slot).start()


def kernel(b_input, adj, a_weight, a_bias):
    bias2d = a_bias.reshape(1, F)
    return pl.pallas_call(
        _fused_kernel,
        grid=(NSTEPS,),
        in_specs=[
            pl.BlockSpec(memory_space=pltpu.MemorySpace.HBM),
            pl.BlockSpec(memory_space=pltpu.MemorySpace.HBM),
            pl.BlockSpec(memory_space=pltpu.MemorySpace.HBM),
            pl.BlockSpec(memory_space=pltpu.MemorySpace.HBM),
        ],
        out_specs=pl.BlockSpec((BM, F), lambda i: (i, 0)),
        out_shape=jax.ShapeDtypeStruct((N, F), jnp.float32),
        scratch_shapes=[
            pltpu.VMEM((NBUF, BM, N), jnp.float32),
            pltpu.VMEM((N, F), jnp.bfloat16),
            pltpu.VMEM((N, F), jnp.float32),
            pltpu.VMEM((F, F), jnp.float32),
            pltpu.VMEM((1, F), jnp.float32),
            pltpu.SemaphoreType.DMA((NBUF,)),
            pltpu.SemaphoreType.DMA((3,)),
        ],
    )(b_input, adj, a_weight, bias2d)
